# Initial kernel scaffold; baseline (speedup 1.0000x reference)
import jax
import jax.numpy as jnp
from jax.experimental import pallas as pl


def kernel(X, gamma, beta, root_nid):
    raise NotImplementedError("write your pallas kernel here")


# speedup vs baseline: 1.8519x; 1.8519x over previous
import functools

import jax
import jax.numpy as jnp
from jax import lax
from jax.experimental import pallas as pl
from jax.experimental.pallas import tpu as pltpu
from jax.experimental.pallas import tpu_sc as plsc

N_NODES = 100000
N_ROOT = 50000
D_FEAT = 128
EPS = 1e-5

NC = 2
NS = 16
NW = NC * NS

CH = 1568
K = 112
NCH = CH // K
G = D_FEAT // 16


def _sc_stats_body(x_hbm, root_hbm, part_hbm, idxb, buf, acc_v, isem, sem0, sem1):
    wid = lax.axis_index("c") * NS + lax.axis_index("s")
    start = wid * CH
    base = jnp.minimum(start, N_ROOT - CH)
    lo = start - base

    idx_cps = [
        pltpu.make_async_copy(root_hbm.at[pl.ds(base + c * K, K)], idxb.at[c], isem)
        for c in range(NCH)
    ]
    for cp in idx_cps:
        cp.start()
    for cp in idx_cps:
        cp.wait()

    sems = [sem0, sem1]
    row_cps = [
        pltpu.make_async_copy(x_hbm.at[idxb.at[c]], buf.at[c % 2], sems[c % 2])
        for c in range(NCH)
    ]
    row_cps[0].start()

    accs = tuple(jnp.zeros((16,), jnp.float32) for _ in range(2 * G))
    for c in range(NCH):
        b = c % 2
        if c + 1 < NCH:
            row_cps[c + 1].start()
        row_cps[c].wait()

        jlo = jnp.clip(lo - c * K, 0, K)

        def zero_body(j, _):
            for g in range(G):
                buf[b, j, pl.ds(g * 16, 16)] = jnp.zeros((16,), jnp.float32)
            return 0

        lax.fori_loop(0, jlo, zero_body, 0)

        def acc_body(j, carry):
            vals = [buf[b, j, pl.ds(g * 16, 16)] for g in range(G)]
            s = tuple(carry[g] + vals[g] for g in range(G))
            s2 = tuple(carry[G + g] + vals[g] * vals[g] for g in range(G))
            return s + s2

        accs = lax.fori_loop(0, K, acc_body, accs)

    for g in range(2 * G):
        acc_v[pl.ds(g * 16, 16)] = accs[g]
    pltpu.sync_copy(acc_v, part_hbm.at[wid])


@functools.cache
def _sc_stats():
    return pl.kernel(
        _sc_stats_body,
        out_type=jax.ShapeDtypeStruct((NW, 2 * D_FEAT), jnp.float32),
        mesh=plsc.VectorSubcoreMesh(
            core_axis_name="c", subcore_axis_name="s", num_cores=NC, num_subcores=NS
        ),
        scratch_types=[
            pltpu.VMEM((NCH, K), jnp.int32),
            pltpu.VMEM((2, K, D_FEAT), jnp.float32),
            pltpu.VMEM((2 * D_FEAT,), jnp.float32),
            pltpu.SemaphoreType.DMA,
            pltpu.SemaphoreType.DMA,
            pltpu.SemaphoreType.DMA,
        ],
    )


def _tc_norm_body(part_ref, gamma_ref, beta_ref, x_ref, o_ref):
    part = part_ref[...]
    s = jnp.sum(part[:, :D_FEAT], axis=0, keepdims=True)
    s2 = jnp.sum(part[:, D_FEAT:], axis=0, keepdims=True)
    inv_n = 1.0 / N_ROOT
    mean = s * inv_n
    var = s2 * inv_n - mean * mean
    rstd = lax.rsqrt(var + EPS)
    scale = gamma_ref[...] * rstd
    shift = beta_ref[...] - mean * scale
    o_ref[...] = x_ref[...] * scale + shift


_BLK = 4000


@functools.partial(jax.jit, donate_argnums=())
def _tc_norm(partials, gamma2d, beta2d, X):
    grid = N_NODES // _BLK
    return pl.pallas_call(
        _tc_norm_body,
        grid=(grid,),
        in_specs=[
            pl.BlockSpec((NW, 2 * D_FEAT), lambda i: (0, 0)),
            pl.BlockSpec((1, D_FEAT), lambda i: (0, 0)),
            pl.BlockSpec((1, D_FEAT), lambda i: (0, 0)),
            pl.BlockSpec((_BLK, D_FEAT), lambda i: (i, 0)),
        ],
        out_specs=pl.BlockSpec((_BLK, D_FEAT), lambda i: (i, 0)),
        out_shape=jax.ShapeDtypeStruct((N_NODES, D_FEAT), jnp.float32),
        compiler_params=pltpu.CompilerParams(
            dimension_semantics=("arbitrary",),
        ),
    )(partials, gamma2d, beta2d, X)


def kernel(X, gamma, beta, root_nid):
    partials = _sc_stats()(X, root_nid)
    return _tc_norm(partials, gamma.reshape(1, D_FEAT), beta.reshape(1, D_FEAT), X)


# speedup vs baseline: 2.0544x; 1.1093x over previous
import functools

import jax
import jax.numpy as jnp
from jax import lax
from jax.experimental import pallas as pl
from jax.experimental.pallas import tpu as pltpu
from jax.experimental.pallas import tpu_sc as plsc

N_NODES = 100000
N_ROOT = 50000
D_FEAT = 128
EPS = 1e-5

NC = 2
NS = 16
NW = NC * NS

CH = 1568
K = 112
NCH = CH // K
NB = 4
G = D_FEAT // 16


def _sc_stats_body(
    x_hbm, root_hbm, part_hbm, idxb, buf, acc_v, isem, sem0, sem1, sem2, sem3
):
    wid = lax.axis_index("c") * NS + lax.axis_index("s")
    start = wid * CH
    base = jnp.minimum(start, N_ROOT - CH)
    lo = start - base

    idx_cps = [
        pltpu.make_async_copy(root_hbm.at[pl.ds(base + c * K, K)], idxb.at[c], isem)
        for c in range(NCH)
    ]
    for cp in idx_cps:
        cp.start()
    for cp in idx_cps:
        cp.wait()

    sems = [sem0, sem1, sem2, sem3]
    row_cps = [
        pltpu.make_async_copy(x_hbm.at[idxb.at[c]], buf.at[c % NB], sems[c % NB])
        for c in range(NCH)
    ]
    for c in range(NB - 1):
        row_cps[c].start()

    accs = tuple(jnp.zeros((16,), jnp.float32) for _ in range(2 * G))
    for c in range(NCH):
        b = c % NB
        if c + NB - 1 < NCH:
            row_cps[c + NB - 1].start()
        row_cps[c].wait()

        jlo = jnp.clip(lo - c * K, 0, K)

        def zero_body(j, _):
            for g in range(G):
                buf[b, j, pl.ds(g * 16, 16)] = jnp.zeros((16,), jnp.float32)
            return 0

        lax.fori_loop(0, jlo, zero_body, 0)

        def acc_body(j, carry):
            vals = [buf[b, j, pl.ds(g * 16, 16)] for g in range(G)]
            s = tuple(carry[g] + vals[g] for g in range(G))
            s2 = tuple(carry[G + g] + vals[g] * vals[g] for g in range(G))
            return s + s2

        accs = lax.fori_loop(0, K, acc_body, accs)

    for g in range(2 * G):
        acc_v[pl.ds(g * 16, 16)] = accs[g]
    pltpu.sync_copy(acc_v, part_hbm.at[wid])


@functools.cache
def _sc_stats():
    return pl.kernel(
        _sc_stats_body,
        out_type=jax.ShapeDtypeStruct((NW, 2 * D_FEAT), jnp.float32),
        mesh=plsc.VectorSubcoreMesh(
            core_axis_name="c", subcore_axis_name="s", num_cores=NC, num_subcores=NS
        ),
        scratch_types=[
            pltpu.VMEM((NCH, K), jnp.int32),
            pltpu.VMEM((NB, K, D_FEAT), jnp.float32),
            pltpu.VMEM((2 * D_FEAT,), jnp.float32),
            pltpu.SemaphoreType.DMA,
            pltpu.SemaphoreType.DMA,
            pltpu.SemaphoreType.DMA,
            pltpu.SemaphoreType.DMA,
            pltpu.SemaphoreType.DMA,
        ],
    )


def _tc_norm_body(part_ref, gamma_ref, beta_ref, x_ref, o_ref):
    part = part_ref[...]
    s = jnp.sum(part[:, :D_FEAT], axis=0, keepdims=True)
    s2 = jnp.sum(part[:, D_FEAT:], axis=0, keepdims=True)
    inv_n = 1.0 / N_ROOT
    mean = s * inv_n
    var = s2 * inv_n - mean * mean
    rstd = lax.rsqrt(var + EPS)
    scale = gamma_ref[...] * rstd
    shift = beta_ref[...] - mean * scale
    o_ref[...] = x_ref[...] * scale + shift


_BLK = 10000


@functools.partial(jax.jit, donate_argnums=())
def _tc_norm(partials, gamma2d, beta2d, X):
    grid = N_NODES // _BLK
    return pl.pallas_call(
        _tc_norm_body,
        grid=(grid,),
        in_specs=[
            pl.BlockSpec((NW, 2 * D_FEAT), lambda i: (0, 0)),
            pl.BlockSpec((1, D_FEAT), lambda i: (0, 0)),
            pl.BlockSpec((1, D_FEAT), lambda i: (0, 0)),
            pl.BlockSpec((_BLK, D_FEAT), lambda i: (i, 0)),
        ],
        out_specs=pl.BlockSpec((_BLK, D_FEAT), lambda i: (i, 0)),
        out_shape=jax.ShapeDtypeStruct((N_NODES, D_FEAT), jnp.float32),
        compiler_params=pltpu.CompilerParams(
            dimension_semantics=("arbitrary",),
        ),
    )(partials, gamma2d, beta2d, X)


def kernel(X, gamma, beta, root_nid):
    partials = _sc_stats()(X, root_nid)
    return _tc_norm(partials, gamma.reshape(1, D_FEAT), beta.reshape(1, D_FEAT), X)


# speedup vs baseline: 2.0998x; 1.0221x over previous
import functools

import jax
import jax.numpy as jnp
from jax import lax
from jax.experimental import pallas as pl
from jax.experimental.pallas import tpu as pltpu
from jax.experimental.pallas import tpu_sc as plsc

N_NODES = 100000
N_ROOT = 50000
D_FEAT = 128
EPS = 1e-5

NC = 2
NS = 16
NW = NC * NS

CH = 1568
K = 112
NCH = CH // K
NB = 4
G = D_FEAT // 16


def _sc_stats_body(
    x_hbm, root_hbm, part_hbm, idxb, buf, acc_v, isem, sem0, sem1, sem2, sem3
):
    wid = lax.axis_index("c") * NS + lax.axis_index("s")
    start = wid * CH
    base = jnp.minimum(start, N_ROOT - CH)
    lo = start - base

    idx_cps = [
        pltpu.make_async_copy(root_hbm.at[pl.ds(base + c * K, K)], idxb.at[c], isem)
        for c in range(NCH)
    ]
    for cp in idx_cps:
        cp.start()
    for cp in idx_cps:
        cp.wait()

    sems = [sem0, sem1, sem2, sem3]
    row_cps = [
        pltpu.make_async_copy(x_hbm.at[idxb.at[c]], buf.at[c % NB], sems[c % NB])
        for c in range(NCH)
    ]
    for c in range(NB - 1):
        row_cps[c].start()

    accs = tuple(jnp.zeros((16,), jnp.float32) for _ in range(2 * G))
    for c in range(NCH):
        b = c % NB
        if c + NB - 1 < NCH:
            row_cps[c + NB - 1].start()
        row_cps[c].wait()

        jlo = jnp.clip(lo - c * K, 0, K)

        def zero_body(j, _):
            for g in range(G):
                buf[b, j, pl.ds(g * 16, 16)] = jnp.zeros((16,), jnp.float32)
            return 0

        lax.fori_loop(0, jlo, zero_body, 0)

        def acc_body(j, carry):
            vals = [buf[b, j, pl.ds(g * 16, 16)] for g in range(G)]
            s = tuple(carry[g] + vals[g] for g in range(G))
            s2 = tuple(carry[G + g] + vals[g] * vals[g] for g in range(G))
            return s + s2

        accs = lax.fori_loop(0, K, acc_body, accs)

    for g in range(2 * G):
        acc_v[pl.ds(g * 16, 16)] = accs[g]
    pltpu.sync_copy(acc_v, part_hbm.at[wid])


@functools.cache
def _sc_stats():
    return pl.kernel(
        _sc_stats_body,
        out_type=jax.ShapeDtypeStruct((NW, 2 * D_FEAT), jnp.float32),
        mesh=plsc.VectorSubcoreMesh(
            core_axis_name="c", subcore_axis_name="s", num_cores=NC, num_subcores=NS
        ),
        scratch_types=[
            pltpu.VMEM((NCH, K), jnp.int32),
            pltpu.VMEM((NB, K, D_FEAT), jnp.float32),
            pltpu.VMEM((2 * D_FEAT,), jnp.float32),
            pltpu.SemaphoreType.DMA,
            pltpu.SemaphoreType.DMA,
            pltpu.SemaphoreType.DMA,
            pltpu.SemaphoreType.DMA,
            pltpu.SemaphoreType.DMA,
        ],
    )


def _tc_norm_body(part_ref, gamma_ref, beta_ref, x_ref, o_ref):
    part = part_ref[...]
    s = jnp.sum(part[:, :D_FEAT], axis=0, keepdims=True)
    s2 = jnp.sum(part[:, D_FEAT:], axis=0, keepdims=True)
    inv_n = 1.0 / N_ROOT
    mean = s * inv_n
    var = s2 * inv_n - mean * mean
    rstd = lax.rsqrt(var + EPS)
    scale = gamma_ref[...] * rstd
    shift = beta_ref[...] - mean * scale
    o_ref[...] = x_ref[...] * scale + shift


_BLK = 20000


@functools.partial(jax.jit, donate_argnums=())
def _tc_norm(partials, gamma2d, beta2d, X):
    grid = N_NODES // _BLK
    return pl.pallas_call(
        _tc_norm_body,
        grid=(grid,),
        in_specs=[
            pl.BlockSpec((NW, 2 * D_FEAT), lambda i: (0, 0)),
            pl.BlockSpec((1, D_FEAT), lambda i: (0, 0)),
            pl.BlockSpec((1, D_FEAT), lambda i: (0, 0)),
            pl.BlockSpec((_BLK, D_FEAT), lambda i: (i, 0)),
        ],
        out_specs=pl.BlockSpec((_BLK, D_FEAT), lambda i: (i, 0)),
        out_shape=jax.ShapeDtypeStruct((N_NODES, D_FEAT), jnp.float32),
        compiler_params=pltpu.CompilerParams(
            dimension_semantics=("arbitrary",),
        ),
    )(partials, gamma2d, beta2d, X)


def kernel(X, gamma, beta, root_nid):
    partials = _sc_stats()(X, root_nid)
    return _tc_norm(partials, gamma.reshape(1, D_FEAT), beta.reshape(1, D_FEAT), X)


# speedup vs baseline: 2.1047x; 1.0023x over previous
import functools

import jax
import jax.numpy as jnp
from jax import lax
from jax.experimental import pallas as pl
from jax.experimental.pallas import tpu as pltpu
from jax.experimental.pallas import tpu_sc as plsc

N_NODES = 100000
N_ROOT = 50000
D_FEAT = 128
EPS = 1e-5

NC = 2
NS = 16
NW = NC * NS

CH = 1568
K = 112
NCH = CH // K
NB = 7
G = D_FEAT // 16


def _sc_stats_body(x_hbm, root_hbm, part_hbm, idxb, buf, acc_v, isem, *sems):
    wid = lax.axis_index("c") * NS + lax.axis_index("s")
    start = wid * CH
    base = jnp.minimum(start, N_ROOT - CH)
    lo = start - base

    idx_cps = [
        pltpu.make_async_copy(root_hbm.at[pl.ds(base + c * K, K)], idxb.at[c], isem)
        for c in range(NCH)
    ]
    for cp in idx_cps:
        cp.start()
    for cp in idx_cps:
        cp.wait()

    row_cps = [
        pltpu.make_async_copy(x_hbm.at[idxb.at[c]], buf.at[c % NB], sems[c % NB])
        for c in range(NCH)
    ]
    for c in range(NB - 1):
        row_cps[c].start()

    accs = tuple(jnp.zeros((16,), jnp.float32) for _ in range(2 * G))
    for c in range(NCH):
        b = c % NB
        if c + NB - 1 < NCH:
            row_cps[c + NB - 1].start()
        row_cps[c].wait()

        jlo = jnp.clip(lo - c * K, 0, K)

        def zero_body(j, _):
            for g in range(G):
                buf[b, j, pl.ds(g * 16, 16)] = jnp.zeros((16,), jnp.float32)
            return 0

        lax.fori_loop(0, jlo, zero_body, 0)

        def acc_body(j, carry):
            vals = [buf[b, j, pl.ds(g * 16, 16)] for g in range(G)]
            s = tuple(carry[g] + vals[g] for g in range(G))
            s2 = tuple(carry[G + g] + vals[g] * vals[g] for g in range(G))
            return s + s2

        accs = lax.fori_loop(0, K, acc_body, accs)

    for g in range(2 * G):
        acc_v[pl.ds(g * 16, 16)] = accs[g]
    pltpu.sync_copy(acc_v, part_hbm.at[wid])


@functools.cache
def _sc_stats():
    return pl.kernel(
        _sc_stats_body,
        out_type=jax.ShapeDtypeStruct((NW, 2 * D_FEAT), jnp.float32),
        mesh=plsc.VectorSubcoreMesh(
            core_axis_name="c", subcore_axis_name="s", num_cores=NC, num_subcores=NS
        ),
        scratch_types=[
            pltpu.VMEM((NCH, K), jnp.int32),
            pltpu.VMEM((NB, K, D_FEAT), jnp.float32),
            pltpu.VMEM((2 * D_FEAT,), jnp.float32),
        ]
        + [pltpu.SemaphoreType.DMA] * (NB + 1),
    )


def _tc_norm_body(part_ref, gamma_ref, beta_ref, x_ref, o_ref):
    part = part_ref[...]
    s = jnp.sum(part[:, :D_FEAT], axis=0, keepdims=True)
    s2 = jnp.sum(part[:, D_FEAT:], axis=0, keepdims=True)
    inv_n = 1.0 / N_ROOT
    mean = s * inv_n
    var = s2 * inv_n - mean * mean
    rstd = lax.rsqrt(var + EPS)
    scale = gamma_ref[...] * rstd
    shift = beta_ref[...] - mean * scale
    o_ref[...] = x_ref[...] * scale + shift


_BLK = 20000


@functools.partial(jax.jit, donate_argnums=())
def _tc_norm(partials, gamma2d, beta2d, X):
    grid = N_NODES // _BLK
    return pl.pallas_call(
        _tc_norm_body,
        grid=(grid,),
        in_specs=[
            pl.BlockSpec((NW, 2 * D_FEAT), lambda i: (0, 0)),
            pl.BlockSpec((1, D_FEAT), lambda i: (0, 0)),
            pl.BlockSpec((1, D_FEAT), lambda i: (0, 0)),
            pl.BlockSpec((_BLK, D_FEAT), lambda i: (i, 0)),
        ],
        out_specs=pl.BlockSpec((_BLK, D_FEAT), lambda i: (i, 0)),
        out_shape=jax.ShapeDtypeStruct((N_NODES, D_FEAT), jnp.float32),
        compiler_params=pltpu.CompilerParams(
            dimension_semantics=("arbitrary",),
        ),
    )(partials, gamma2d, beta2d, X)


def kernel(X, gamma, beta, root_nid):
    partials = _sc_stats()(X, root_nid)
    return _tc_norm(partials, gamma.reshape(1, D_FEAT), beta.reshape(1, D_FEAT), X)
